# baseline (speedup 1.0000x reference)
import jax
import jax.numpy as jnp
from jax.experimental import pallas as pl
from jax.experimental.pallas import tpu as pltpu

_NBUF = 4
_OBUF = 2


def _encode_kernel(x_hbm, w_ref, b_ref, pos_ref, o_hbm, xbuf, obuf, xsems, osems):
    i = pl.program_id(0)
    nsteps = pl.num_programs(0)

    @pl.when(i == 0)
    def _warmup():
        for k in range(_NBUF):
            pltpu.make_async_copy(x_hbm.at[k], xbuf.at[k], xsems.at[k]).start()

    slot = jax.lax.rem(i, _NBUF)
    pltpu.make_async_copy(x_hbm.at[i], xbuf.at[slot], xsems.at[slot]).wait()

    oslot = jax.lax.rem(i, _OBUF)

    @pl.when(i >= _OBUF)
    def _wait_store():
        pltpu.make_async_copy(
            obuf.at[oslot], o_hbm.at[i - _OBUF], osems.at[oslot]
        ).wait()

    acc = jax.lax.dot_general(
        xbuf[slot], w_ref[...], (((1,), (1,)), ((), ())),
        preferred_element_type=jnp.float32,
    )
    obuf[oslot] = acc + b_ref[...] + pos_ref[...]
    pltpu.make_async_copy(obuf.at[oslot], o_hbm.at[i], osems.at[oslot]).start()

    nxt = i + _NBUF
    nslot = jax.lax.rem(nxt, _NBUF)

    @pl.when(nxt < nsteps)
    def _prefetch():
        pltpu.make_async_copy(x_hbm.at[nxt], xbuf.at[nslot], xsems.at[nslot]).start()

    @pl.when(i == nsteps - 1)
    def _drain():
        for k in range(_OBUF):
            s = jax.lax.rem(i - k + _OBUF, _OBUF)
            pltpu.make_async_copy(
                obuf.at[s], o_hbm.at[i - k], osems.at[s]
            ).wait()


def kernel(patch, W, b, pos_table):
    B, N, D = patch.shape
    P = W.shape[0]
    b2 = b.reshape(1, P)
    return pl.pallas_call(
        _encode_kernel,
        grid=(B,),
        in_specs=[
            pl.BlockSpec(memory_space=pltpu.HBM),
            pl.BlockSpec((P, D), lambda i: (0, 0)),
            pl.BlockSpec((1, P), lambda i: (0, 0)),
            pl.BlockSpec((N, P), lambda i: (0, 0)),
        ],
        out_specs=pl.BlockSpec(memory_space=pltpu.HBM),
        out_shape=jax.ShapeDtypeStruct((B, N, P), jnp.float32),
        scratch_shapes=[
            pltpu.VMEM((_NBUF, N, D), jnp.float32),
            pltpu.VMEM((_OBUF, N, P), jnp.float32),
            pltpu.SemaphoreType.DMA((_NBUF,)),
            pltpu.SemaphoreType.DMA((_OBUF,)),
        ],
        compiler_params=pltpu.CompilerParams(
            dimension_semantics=("arbitrary",),
        ),
    )(patch, W, b2, pos_table)
